# Initial kernel scaffold; baseline (speedup 1.0000x reference)
import jax
import jax.numpy as jnp
from jax.experimental import pallas as pl


def kernel(x, edge_index, W1, b1, W2, b2):
    raise NotImplementedError("write your pallas kernel here")


# speedup vs baseline: 22.9582x; 22.9582x over previous
import functools

import jax
import jax.numpy as jnp
from jax import lax
from jax.experimental import pallas as pl
from jax.experimental.pallas import tpu as pltpu
from jax.experimental.pallas import tpu_sc as plsc

NC = 2
NS = 16
CHUNK = 128
DEGW = 8


def _make_agg(n_pad, d, n_chunks):
  mesh = plsc.VectorSubcoreMesh(core_axis_name="c", subcore_axis_name="s")
  rows_per_tile = n_pad // NS

  @functools.partial(
      pl.kernel,
      mesh=mesh,
      out_type=jax.ShapeDtypeStruct((NC, n_pad, d), jnp.float32),
      compiler_params=pltpu.CompilerParams(use_tc_tiling_on_sc=False),
      scratch_types=[
          pltpu.VMEM((n_chunks, CHUNK), jnp.int32),
          pltpu.VMEM((n_chunks, CHUNK), jnp.int32),
          pltpu.VMEM((CHUNK, d), jnp.float32),
          pltpu.VMEM_SHARED((n_pad, d), jnp.float32),
          pltpu.SemaphoreType.DMA,
      ],
  )
  def agg(hs_hbm, src_hbm, dst_hbm, zeros_hbm, out_hbm,
          src_v, dst_v, rows_v, acc, sem):
    c = lax.axis_index("c")
    s = lax.axis_index("s")
    tid = c * NS + s
    pltpu.sync_copy(src_hbm.at[tid], src_v)
    pltpu.sync_copy(dst_hbm.at[tid], dst_v)
    sl = pl.ds(s * rows_per_tile, rows_per_tile)
    pltpu.sync_copy(zeros_hbm.at[sl], acc.at[sl])
    plsc.subcore_barrier()

    def body(j, carry):
      pltpu.async_copy(hs_hbm.at[src_v.at[j]], rows_v, sem).wait()
      pltpu.sync_copy(rows_v, acc.at[dst_v.at[j]], add=True)
      return carry

    lax.fori_loop(0, n_chunks, body, 0)
    plsc.subcore_barrier()
    pltpu.sync_copy(acc.at[sl], out_hbm.at[c, sl])

  return agg


def _make_deg(n_pad, n_chunks, w=DEGW):
  mesh = plsc.VectorSubcoreMesh(core_axis_name="c", subcore_axis_name="s")
  rows_per_tile = n_pad // NS

  @functools.partial(
      pl.kernel,
      mesh=mesh,
      out_type=jax.ShapeDtypeStruct((NC, n_pad, w), jnp.float32),
      compiler_params=pltpu.CompilerParams(use_tc_tiling_on_sc=False),
      scratch_types=[
          pltpu.VMEM((n_chunks, CHUNK), jnp.int32),
          pltpu.VMEM((CHUNK, w), jnp.float32),
          pltpu.VMEM_SHARED((n_pad, w), jnp.float32),
      ],
  )
  def deg(dst_hbm, ones_hbm, zeros_hbm, out_hbm, dst_v, ones_v, acc):
    c = lax.axis_index("c")
    s = lax.axis_index("s")
    tid = c * NS + s
    pltpu.sync_copy(dst_hbm.at[tid], dst_v)
    pltpu.sync_copy(ones_hbm, ones_v)
    sl = pl.ds(s * rows_per_tile, rows_per_tile)
    pltpu.sync_copy(zeros_hbm.at[pl.ds(0, rows_per_tile)], acc.at[sl])
    plsc.subcore_barrier()

    def body(j, carry):
      pltpu.sync_copy(ones_v, acc.at[dst_v.at[j]], add=True)
      return carry

    lax.fori_loop(0, n_chunks, body, 0)
    plsc.subcore_barrier()
    pltpu.sync_copy(acc.at[sl], out_hbm.at[c, sl])

  return deg


def _tc_first(x, w1, dega, degb, block):
  n = x.shape[0]
  d_in, d_hid = w1.shape

  def body(x_ref, w_ref, da_ref, db_ref, h_ref, hs_ref, dis_ref):
    deg = da_ref[...] + db_ref[...] + 1.0
    dis = lax.rsqrt(deg)
    h = jnp.dot(x_ref[...], w_ref[...], preferred_element_type=jnp.float32)
    h_ref[...] = h
    hs_ref[...] = h * dis
    dis_ref[...] = dis

  grid = (n // block,)
  return pl.pallas_call(
      body,
      grid=grid,
      in_specs=[
          pl.BlockSpec((block, d_in), lambda i: (i, 0)),
          pl.BlockSpec((d_in, d_hid), lambda i: (0, 0)),
          pl.BlockSpec((block, 1), lambda i: (i, 0)),
          pl.BlockSpec((block, 1), lambda i: (i, 0)),
      ],
      out_specs=[
          pl.BlockSpec((block, d_hid), lambda i: (i, 0)),
          pl.BlockSpec((block, d_hid), lambda i: (i, 0)),
          pl.BlockSpec((block, 1), lambda i: (i, 0)),
      ],
      out_shape=[
          jax.ShapeDtypeStruct((n, d_hid), jnp.float32),
          jax.ShapeDtypeStruct((n, d_hid), jnp.float32),
          jax.ShapeDtypeStruct((n, 1), jnp.float32),
      ],
  )(x, w1, dega, degb)


def _tc_mid(pa, pb, h1, dis, b1, w2, block):
  n, d_hid = h1.shape
  d_out = w2.shape[1]

  def body(pa_ref, pb_ref, h_ref, dis_ref, b_ref, w_ref, h2_ref, hs2_ref):
    dis = dis_ref[...]
    z = dis * (pa_ref[...] + pb_ref[...]) + (dis * dis) * h_ref[...] + b_ref[...]
    z = jnp.maximum(z, 0.0)
    h2 = jnp.dot(z, w_ref[...], preferred_element_type=jnp.float32)
    h2_ref[...] = h2
    hs2_ref[...] = h2 * dis

  grid = (n // block,)
  return pl.pallas_call(
      body,
      grid=grid,
      in_specs=[
          pl.BlockSpec((block, d_hid), lambda i: (i, 0)),
          pl.BlockSpec((block, d_hid), lambda i: (i, 0)),
          pl.BlockSpec((block, d_hid), lambda i: (i, 0)),
          pl.BlockSpec((block, 1), lambda i: (i, 0)),
          pl.BlockSpec((1, d_hid), lambda i: (0, 0)),
          pl.BlockSpec((d_hid, d_out), lambda i: (0, 0)),
      ],
      out_specs=[
          pl.BlockSpec((block, d_out), lambda i: (i, 0)),
          pl.BlockSpec((block, d_out), lambda i: (i, 0)),
      ],
      out_shape=[
          jax.ShapeDtypeStruct((n, d_out), jnp.float32),
          jax.ShapeDtypeStruct((n, d_out), jnp.float32),
      ],
  )(pa, pb, h1, dis, b1, w2)


def _tc_last(qa, qb, h2, dis, b2, block):
  n, d_out = h2.shape

  def body(qa_ref, qb_ref, h_ref, dis_ref, b_ref, o_ref):
    dis = dis_ref[...]
    o_ref[...] = (dis * (qa_ref[...] + qb_ref[...])
                  + (dis * dis) * h_ref[...] + b_ref[...])

  grid = (n // block,)
  return pl.pallas_call(
      body,
      grid=grid,
      in_specs=[
          pl.BlockSpec((block, d_out), lambda i: (i, 0)),
          pl.BlockSpec((block, d_out), lambda i: (i, 0)),
          pl.BlockSpec((block, d_out), lambda i: (i, 0)),
          pl.BlockSpec((block, 1), lambda i: (i, 0)),
          pl.BlockSpec((1, d_out), lambda i: (0, 0)),
      ],
      out_specs=pl.BlockSpec((block, d_out), lambda i: (i, 0)),
      out_shape=jax.ShapeDtypeStruct((n, d_out), jnp.float32),
  )(qa, qb, h2, dis, b2)


def kernel(x, edge_index, W1, b1, W2, b2):
  n, d_in = x.shape
  d_hid = W1.shape[1]
  d_out = W2.shape[1]
  e = edge_index.shape[1]

  src = edge_index[0].astype(jnp.int32)
  dst = edge_index[1].astype(jnp.int32)

  n_tiles = NC * NS
  n_chunks = -(-e // (n_tiles * CHUNK))
  e_pad = n_tiles * n_chunks * CHUNK
  n_pad = -(-(n + 1) // (NS * 8)) * (NS * 8)

  src_p = jnp.concatenate([src, jnp.zeros((e_pad - e,), jnp.int32)])
  dst_p = jnp.concatenate([dst, jnp.full((e_pad - e,), n, jnp.int32)])
  src_arr = src_p.reshape(n_tiles, n_chunks, CHUNK)
  dst_arr = dst_p.reshape(n_tiles, n_chunks, CHUNK)

  zeros_big = jnp.zeros((n_pad, max(d_hid, d_out)), jnp.float32)
  ones_col = jnp.ones((CHUNK, DEGW), jnp.float32)
  zeros_col = jnp.zeros((n_pad, DEGW), jnp.float32)

  deg2 = _make_deg(n_pad, n_chunks)(dst_arr, ones_col, zeros_col)
  dega = deg2[0, :n, :1]
  degb = deg2[1, :n, :1]

  block = 2000 if n % 2000 == 0 else n
  h1, hs1, dis = _tc_first(x, W1, dega, degb, block)

  agg1 = _make_agg(n_pad, d_hid, n_chunks)(
      hs1, src_arr, dst_arr, zeros_big[:, :d_hid])
  h2, hs2 = _tc_mid(agg1[0, :n], agg1[1, :n], h1, dis,
                    b1.reshape(1, d_hid), W2, block)

  agg2 = _make_agg(n_pad, d_out, n_chunks)(
      hs2, src_arr, dst_arr, zeros_big[:, :d_out])
  out = _tc_last(agg2[0, :n], agg2[1, :n], h2, dis,
                 b2.reshape(1, d_out), block)
  return out
